# Initial kernel scaffold; baseline (speedup 1.0000x reference)
import jax
import jax.numpy as jnp
from jax.experimental import pallas as pl


def kernel(sentences, W_ih_f, W_hh_f, b_ih_f, b_hh_f, W_ih_b, W_hh_b, b_ih_b, b_hh_b):
    raise NotImplementedError("write your pallas kernel here")


# speedup vs baseline: 8.7516x; 8.7516x over previous
import math

import jax
import jax.numpy as jnp
from jax import lax
from jax.experimental import pallas as pl
from jax.experimental.pallas import tpu as pltpu

T, B, I, H = 128, 128, 512, 512
KPAD = 8


def _bilstm_kernel(xf_ref, xb_ref, wih_f_ref, whh_f_ref, bf_ref,
                   wih_b_ref, whh_b_ref, bb_ref,
                   outf_ref, outb_ref, hf, cf, hb, cb):
    t = pl.program_id(0)

    @pl.when(t == 0)
    def _init():
        hf[...] = jnp.zeros_like(hf)
        cf[...] = jnp.zeros_like(cf)
        hb[...] = jnp.zeros_like(hb)
        cb[...] = jnp.zeros_like(cb)

    def _step(x, wih_ref, whh_ref, b_ref, h, c, out_ref):
        g = (jnp.dot(x, wih_ref[...], preferred_element_type=jnp.float32)
             + jnp.dot(h[...], whh_ref[...], preferred_element_type=jnp.float32)
             + b_ref[...])
        ig = jax.nn.sigmoid(g[:, 0:H])
        fg = jax.nn.sigmoid(g[:, H:2 * H])
        gg = jnp.tanh(g[:, 2 * H:3 * H])
        og = jax.nn.sigmoid(g[:, 3 * H:4 * H])
        c_new = fg * c[...] + ig * gg
        h_new = og * jnp.tanh(c_new)
        c[...] = c_new
        h[...] = h_new
        out_ref[0] = h_new

    _step(xf_ref[0], wih_f_ref, whh_f_ref, bf_ref, hf, cf, outf_ref)
    _step(xb_ref[0], wih_b_ref, whh_b_ref, bb_ref, hb, cb, outb_ref)


def _align_topk_kernel(f_ref, b_ref, lstm_ref, vals_ref, idx_ref):
    lf = f_ref[0]
    lb = b_ref[0]
    lstm_ref[0, :, 0:H] = lf
    lstm_ref[0, :, H:2 * H] = lb
    dn = (((1,), (1,)), ((), ()))
    a = (lax.dot_general(lf, lf, dn, preferred_element_type=jnp.float32)
         + lax.dot_general(lb, lb, dn, preferred_element_type=jnp.float32))
    a = a * (1.0 / math.sqrt(2 * H))
    iota = lax.broadcasted_iota(jnp.int32, (T, T), 1)
    neg = jnp.float32(-3e38)
    vals, idxs = [], []
    for _ in range(3):
        m = jnp.max(a, axis=1, keepdims=True)
        sel = jnp.where(a == m, iota, T)
        ix = jnp.min(sel, axis=1, keepdims=True)
        vals.append(m)
        idxs.append(ix)
        a = jnp.where(iota == ix, neg, a)
    col = lax.broadcasted_iota(jnp.int32, (T, KPAD), 1)
    v = jnp.where(col == 0, vals[0],
                  jnp.where(col == 1, vals[1],
                            jnp.where(col == 2, vals[2], 0.0)))
    ii = jnp.where(col == 0, idxs[0],
                   jnp.where(col == 1, idxs[1],
                             jnp.where(col == 2, idxs[2], 0)))
    vals_ref[0] = v
    idx_ref[0] = ii


def kernel(sentences, W_ih_f, W_hh_f, b_ih_f, b_hh_f,
           W_ih_b, W_hh_b, b_ih_b, b_hh_b):
    wih_f = W_ih_f.T
    whh_f = W_hh_f.T
    wih_b = W_ih_b.T
    whh_b = W_hh_b.T
    bias_f = (b_ih_f + b_hh_f).reshape(1, 4 * H)
    bias_b = (b_ih_b + b_hh_b).reshape(1, 4 * H)

    out_f, out_b = pl.pallas_call(
        _bilstm_kernel,
        grid=(T,),
        in_specs=[
            pl.BlockSpec((1, B, I), lambda t: (t, 0, 0)),
            pl.BlockSpec((1, B, I), lambda t: (T - 1 - t, 0, 0)),
            pl.BlockSpec((I, 4 * H), lambda t: (0, 0)),
            pl.BlockSpec((H, 4 * H), lambda t: (0, 0)),
            pl.BlockSpec((1, 4 * H), lambda t: (0, 0)),
            pl.BlockSpec((I, 4 * H), lambda t: (0, 0)),
            pl.BlockSpec((H, 4 * H), lambda t: (0, 0)),
            pl.BlockSpec((1, 4 * H), lambda t: (0, 0)),
        ],
        out_specs=[
            pl.BlockSpec((1, B, H), lambda t: (t, 0, 0)),
            pl.BlockSpec((1, B, H), lambda t: (T - 1 - t, 0, 0)),
        ],
        out_shape=[
            jax.ShapeDtypeStruct((T, B, H), jnp.float32),
            jax.ShapeDtypeStruct((T, B, H), jnp.float32),
        ],
        scratch_shapes=[pltpu.VMEM((B, H), jnp.float32)] * 4,
        compiler_params=pltpu.CompilerParams(
            dimension_semantics=("arbitrary",),
        ),
    )(sentences, sentences, wih_f, whh_f, bias_f, wih_b, whh_b, bias_b)

    out_f = jnp.transpose(out_f, (1, 0, 2))
    out_b = jnp.transpose(out_b, (1, 0, 2))

    lstm_out, vals, idx = pl.pallas_call(
        _align_topk_kernel,
        grid=(B,),
        in_specs=[
            pl.BlockSpec((1, T, H), lambda b: (b, 0, 0)),
            pl.BlockSpec((1, T, H), lambda b: (b, 0, 0)),
        ],
        out_specs=[
            pl.BlockSpec((1, T, 2 * H), lambda b: (b, 0, 0)),
            pl.BlockSpec((1, T, KPAD), lambda b: (b, 0, 0)),
            pl.BlockSpec((1, T, KPAD), lambda b: (b, 0, 0)),
        ],
        out_shape=[
            jax.ShapeDtypeStruct((B, T, 2 * H), jnp.float32),
            jax.ShapeDtypeStruct((B, T, KPAD), jnp.float32),
            jax.ShapeDtypeStruct((B, T, KPAD), jnp.int32),
        ],
        compiler_params=pltpu.CompilerParams(
            dimension_semantics=("parallel",),
        ),
    )(out_f, out_b)

    adj = idx[:, :, :3].reshape(B, T * 3)
    row1 = jnp.broadcast_to(
        jnp.repeat(jnp.arange(T, dtype=jnp.int32), 3)[None, :], (B, T * 3))
    coo = jnp.stack([adj, row1], axis=1)
    return (coo, vals[:, :, :3], lstm_out)
